# baseline (speedup 1.0000x reference)
import functools

import jax
import jax.numpy as jnp
from jax.experimental import pallas as pl

N = 10000
E = 160000
G_DIM = 256
H1 = 256
H2 = 256
R = 162
NB = 30

BLK = 400


def _final_body(agg2_ref, h_ref, rel_w_ref, rel_b_ref, root2_w_ref, out_ref):
    acc = jnp.dot(agg2_ref[...], rel_w_ref[...], preferred_element_type=jnp.float32)
    acc = acc + jnp.dot(h_ref[...], root2_w_ref[...], preferred_element_type=jnp.float32)
    out_ref[...] = acc + rel_b_ref[...]


def _final_matmuls(agg2, h, rel_w, rel_b, root2_w):
    grid = (N // BLK,)
    return pl.pallas_call(
        _final_body,
        grid=grid,
        in_specs=[
            pl.BlockSpec((BLK, H1), lambda i: (i, 0)),
            pl.BlockSpec((BLK, H1), lambda i: (i, 0)),
            pl.BlockSpec((H1, H2), lambda i: (0, 0)),
            pl.BlockSpec((H2,), lambda i: (0,)),
            pl.BlockSpec((H1, H2), lambda i: (0, 0)),
        ],
        out_specs=pl.BlockSpec((BLK, H2), lambda i: (i, 0)),
        out_shape=jax.ShapeDtypeStruct((N, H2), jnp.float32),
    )(agg2, h, rel_w, rel_b, root2_w)


def kernel(node_features, edge_index, edge_norm, edge_type, comp, basis,
           root1_w, root1_b, rel_w, rel_b, root2_w):
    src = edge_index[0]
    dst = edge_index[1]

    relkey = dst * R + edge_norm
    cnt = jnp.zeros((N * R,), jnp.float32).at[relkey].add(1.0)
    norm = (1.0 / jnp.maximum(cnt, 1.0))[relkey]

    c = jnp.take(comp, edge_norm, axis=0)
    msg = jnp.zeros((E, H1), jnp.float32)
    for b in range(NB):
        xb = node_features @ basis[b]
        msg = msg + c[:, b:b + 1] * jnp.take(xb, src, axis=0)
    msg = msg * norm[:, None]
    agg = jnp.zeros((N, H1), jnp.float32).at[dst].add(msg)
    h = agg + node_features @ root1_w + root1_b

    agg2 = jnp.zeros((N, H1), jnp.float32).at[dst].add(jnp.take(h, src, axis=0))
    return _final_matmuls(agg2, h, rel_w, rel_b, root2_w)


# speedup vs baseline: 2.5612x; 2.5612x over previous
import functools

import jax
import jax.numpy as jnp
from jax import lax
from jax.experimental import pallas as pl
from jax.experimental.pallas import tpu as pltpu
from jax.experimental.pallas import tpu_sc as plsc

N = 10000
E = 160000
G = 256
H1 = 256
H2 = 256
R = 162
NB = 30
NBP = 32

NC = 2
NS = 16
NW = NC * NS
HHALF = 128
XBW = NB * HHALF

ECH = E // 128
EPW3 = (ECH + NS - 1) // NS

KEYS = N * R
KHALF_REAL = KEYS // 2
KSTRIPE = 50688
KPAD = NW * KSTRIPE
KSHIFT = KSTRIPE * NS - KHALF_REAL

STR = 624

BM = 400
CH2 = 1280

_mesh = plsc.VectorSubcoreMesh(core_axis_name="c", subcore_axis_name="s",
                               num_cores=NC, num_subcores=NS)


def _xb_body(x_ref, b_ref, w1_ref, b1_ref, comp_ref, xb_ref, xr_ref,
             compw_ref):
    xb_ref[0] = jnp.dot(x_ref[...], b_ref[0],
                        preferred_element_type=jnp.float32)

    @pl.when(pl.program_id(2) == 0)
    def _():
        xr_ref[0] = jnp.dot(x_ref[...], w1_ref[...],
                            preferred_element_type=jnp.float32) + b1_ref[...]

    @pl.when((pl.program_id(0) == 0) & (pl.program_id(1) == 0)
             & (pl.program_id(2) == 0))
    def _():
        sel = (lax.broadcasted_iota(jnp.int32, (NB, 128), 0)
               == lax.broadcasted_iota(jnp.int32, (NB, 128), 1))
        compw_ref[...] = jnp.dot(comp_ref[...], sel.astype(jnp.float32),
                                 preferred_element_type=jnp.float32)


def _compute_xb(x, basis, root1_w, root1_b, comp):
    return pl.pallas_call(
        _xb_body,
        grid=(NC, N // BM, NB),
        in_specs=[pl.BlockSpec((BM, G), lambda c, i, b: (i, 0)),
                  pl.BlockSpec((1, G, HHALF), lambda c, i, b: (b, 0, c)),
                  pl.BlockSpec((G, HHALF), lambda c, i, b: (0, c)),
                  pl.BlockSpec((1, HHALF), lambda c, i, b: (0, c)),
                  pl.BlockSpec((R, NB), lambda c, i, b: (0, 0))],
        out_specs=[pl.BlockSpec((1, BM, HHALF),
                                lambda c, i, b: (c, i, b)),
                   pl.BlockSpec((1, BM, HHALF), lambda c, i, b: (c, i, 0)),
                   pl.BlockSpec((R, 128), lambda c, i, b: (0, 0))],
        out_shape=[jax.ShapeDtypeStruct((NC, N, XBW), jnp.float32),
                   jax.ShapeDtypeStruct((NC, N, HHALF), jnp.float32),
                   jax.ShapeDtypeStruct((R, 128), jnp.float32)],
    )(x, basis, root1_w, root1_b, comp)


@functools.partial(
    pl.kernel,
    out_type=jax.ShapeDtypeStruct((KPAD,), jnp.float32),
    mesh=_mesh,
    compiler_params=pltpu.CompilerParams(needs_layout_passes=False),
    scratch_types=[
        pltpu.VMEM((KSTRIPE,), jnp.float32),
        pltpu.VMEM((CH2,), jnp.int32),
        pltpu.VMEM((CH2,), jnp.int32),
    ],
)
def _hist_kernel(ef_ref, en_ref, inv_ref, cntb, dstb, relb):
    c = lax.axis_index("c")
    t = lax.axis_index("s")
    w = c * NS + t
    kt0 = w * KSTRIPE
    z16 = jnp.zeros((16,), jnp.float32)
    one16 = jnp.ones((16,), jnp.float32)

    def zf(l, carry):
        cntb[pl.ds(l * 16, 16)] = z16
        return carry
    lax.fori_loop(0, KSTRIPE // 16, zf, 0)

    def chunk(j, carry):
        e0 = j * CH2
        pltpu.sync_copy(ef_ref.at[pl.ds(E + e0, CH2)], dstb)
        pltpu.sync_copy(en_ref.at[pl.ds(e0, CH2)], relb)

        def grp(g, c2):
            d = dstb[pl.ds(g * 16, 16)]
            r = relb[pl.ds(g * 16, 16)]
            k = d * R + r
            pos = k + jnp.where(k >= KHALF_REAL, KSHIFT, 0)
            loc = pos - kt0
            ok = (loc >= 0) & (loc < KSTRIPE)
            loc = jnp.where(ok, loc, 0)
            plsc.addupdate_scatter(cntb, [loc], one16, mask=ok)
            return c2
        lax.fori_loop(0, CH2 // 16, grp, 0)
        return carry
    lax.fori_loop(0, E // CH2, chunk, 0)

    def inv1(l, carry):
        v = cntb[pl.ds(l * 16, 16)]
        cntb[pl.ds(l * 16, 16)] = 1.0 / jnp.maximum(v, 1.0)
        return carry
    lax.fori_loop(0, KSTRIPE // 16, inv1, 0)
    pltpu.sync_copy(cntb, inv_ref.at[pl.ds(kt0, KSTRIPE)])


@functools.partial(
    pl.kernel,
    out_type=jax.ShapeDtypeStruct((E, NBP), jnp.float32),
    mesh=_mesh,
    scratch_types=[
        pltpu.VMEM((128,), jnp.int32),
        pltpu.VMEM((128,), jnp.int32),
        pltpu.VMEM((1, 128), jnp.int32),
        pltpu.VMEM((128,), jnp.float32),
        pltpu.VMEM((128, 128), jnp.float32),
        pltpu.VMEM((128, NBP), jnp.float32),
        pltpu.SemaphoreType.DMA,
    ],
)
def _ce_kernel(ef_ref, en_ref, inv_ref, compw_ref, ce_ref,
               d128, r128, idx, normb, cbufr, cerows, sem0):
    c = lax.axis_index("c")
    t = lax.axis_index("s")

    def chunk(m, carry):
        j = (m * NS + t) * NC + c

        @pl.when(j < ECH)
        def _():
            goff = j * 128
            pltpu.sync_copy(ef_ref.at[pl.ds(E + goff, 128)], d128)
            pltpu.sync_copy(en_ref.at[pl.ds(goff, 128)], r128)
            for l in range(8):
                d = d128[pl.ds(l * 16, 16)]
                r = r128[pl.ds(l * 16, 16)]
                k = d * R + r
                pos = k + jnp.where(k >= KHALF_REAL, KSHIFT, 0)
                idx[0, pl.ds(l * 16, 16)] = pos
            pltpu.async_copy(inv_ref.at[idx.at[0]], normb, sem0).wait()
            for l in range(8):
                idx[0, pl.ds(l * 16, 16)] = r128[pl.ds(l * 16, 16)]
            pltpu.async_copy(compw_ref.at[idx.at[0]], cbufr, sem0).wait()
            for al in range(8):
                nvv = normb[pl.ds(al * 16, 16)]
                for li in range(16):
                    e = al * 16 + li
                    nv = nvv[li]
                    cerows[e, pl.ds(0, 16)] = cbufr[e, pl.ds(0, 16)] * nv
                    cerows[e, pl.ds(16, 16)] = cbufr[e, pl.ds(16, 16)] * nv
            pltpu.sync_copy(cerows, ce_ref.at[pl.ds(goff, 128)])
        return carry
    lax.fori_loop(0, (ECH + NW - 1) // NW, chunk, 0)


@functools.partial(
    pl.kernel,
    out_type=[jax.ShapeDtypeStruct((NC * N, HHALF), jnp.float32),
              jax.ShapeDtypeStruct((NC * N, HHALF), jnp.float32)],
    mesh=_mesh,
    compiler_params=pltpu.CompilerParams(needs_layout_passes=False),
    scratch_types=[
        pltpu.VMEM((128,), jnp.int32),
        pltpu.VMEM((128,), jnp.int32),
        pltpu.VMEM((288,), jnp.int32),
        pltpu.VMEM((4, XBW), jnp.float32),
        pltpu.VMEM((4, XBW), jnp.float32),
        pltpu.VMEM((16, HHALF), jnp.float32),
        pltpu.VMEM((128, NBP), jnp.float32),
        pltpu.VMEM((8, 16), jnp.int32),
        pltpu.SemaphoreType.DMA,
        pltpu.SemaphoreType.DMA,
        pltpu.VMEM_SHARED((N, HHALF), jnp.float32),
    ],
)
def _msg_kernel(xb_ref, ef_ref, ce_ref, xr_ref, hh_ref, a2_ref,
                srcb, dstb, srcb2, xb0, xb1, msgb, crows, sidx,
                sem0, sem1, agg_sh):
    c = lax.axis_index("c")
    t = lax.axis_index("s")
    z16 = jnp.zeros((16,), jnp.float32)
    lanes0 = lax.iota(jnp.int32, 16)
    _qmask = [(lanes0 >= 4 * q) & (lanes0 < 4 * (q + 1)) for q in range(4)]
    rbase = t * STR
    nrow = c * N

    def zero_msgb():
        def zrow(i, carry):
            for v in range(8):
                msgb[i, pl.ds(v * 16, 16)] = z16
            return carry
        lax.fori_loop(0, 16, zrow, 0)

    def zero_agg():
        zero_msgb()

        def zc(i, carry):
            pltpu.sync_copy(msgb, agg_sh.at[pl.ds(rbase + i * 16, 16)])
            return carry
        lax.fori_loop(0, STR // 16, zc, 0)

        @pl.when(t == NS - 1)
        def _():
            pltpu.sync_copy(msgb, agg_sh.at[pl.ds(NS * STR, N - NS * STR)])

    zero_agg()
    plsc.subcore_barrier()

    def stage_edges(j):
        e0 = j * 128
        pltpu.sync_copy(ef_ref.at[pl.ds(e0, 128)], srcb)
        pltpu.sync_copy(ef_ref.at[pl.ds(E + e0, 128)], dstb)
        for l in range(8):
            sv = srcb[pl.ds(l * 16, 16)] + nrow
            srcb[pl.ds(l * 16, 16)] = sv
            sidx[l, :] = dstb[pl.ds(l * 16, 16)]

    def compute4(xb, base):
        def pair(p, c2, _xb=xb):
            for eo in range(2):
                row = 2 * p + eo
                e_lo = base + 2 * p + eo
                mrow = e_lo - (e_lo // 16) * 16
                ca = crows[e_lo, pl.ds(0, 16)]
                cb = crows[e_lo, pl.ds(16, 16)]
                coefs = ([ca[b] for b in range(16)]
                         + [cb[b] for b in range(NB - 16)])

                def vstep(v, c3, _row=row, _e=mrow, _cf=coefs, _x=_xb):
                    acc = _cf[0] * _x[_row, pl.ds(v * 16, 16)]
                    for b in range(1, NB):
                        acc = acc + _cf[b] * _x[_row,
                                                pl.ds(b * 128 + v * 16, 16)]
                    msgb[_e, pl.ds(v * 16, 16)] = acc
                    return c3
                lax.fori_loop(0, 8, vstep, 0)
            return c2
        lax.fori_loop(0, 2, pair, 0)

    def chunk(m, carry):
        j = m * NS + t

        @pl.when(j < ECH)
        def _():
            e0 = j * 128
            stage_edges(j)

            def bperm(l, c2):
                v = srcb[pl.ds(l * 16, 16)]
                for q in range(4):
                    plsc.store_compressed(
                        srcb2.at[pl.ds(l * 32 + q * 8, 16)], v,
                        mask=_qmask[q])
                return c2
            lax.fori_loop(0, 8, bperm, 0)
            pltpu.sync_copy(ce_ref.at[pl.ds(e0, 128)], crows)

            def grp16(g, c2):
                b0 = g * 16
                q0 = g * 32
                d0 = pltpu.async_copy(
                    xb_ref.at[srcb2.at[pl.ds(q0, 4)]], xb0, sem0)
                d1 = pltpu.async_copy(
                    xb_ref.at[srcb2.at[pl.ds(q0 + 8, 4)]], xb1, sem1)
                d0.wait()
                compute4(xb0, b0)
                d2 = pltpu.async_copy(
                    xb_ref.at[srcb2.at[pl.ds(q0 + 16, 4)]], xb0, sem0)
                d1.wait()
                compute4(xb1, b0 + 4)
                d3 = pltpu.async_copy(
                    xb_ref.at[srcb2.at[pl.ds(q0 + 24, 4)]], xb1, sem1)
                d2.wait()
                compute4(xb0, b0 + 8)
                d3.wait()
                compute4(xb1, b0 + 12)
                pltpu.sync_copy(msgb, agg_sh.at[sidx.at[g]], add=True)
                return c2
            lax.fori_loop(0, 8, grp16, 0)
        return carry

    lax.fori_loop(0, EPW3, chunk, 0)

    plsc.subcore_barrier()

    def hpiece(r0):
        pltpu.sync_copy(agg_sh.at[pl.ds(r0, 8)], msgb.at[pl.ds(0, 8)])
        pltpu.sync_copy(xr_ref.at[c, pl.ds(r0, 8)], msgb.at[pl.ds(8, 8)])

        def addr(i, carry):
            for v in range(8):
                a = msgb[i, pl.ds(v * 16, 16)]
                b = msgb[i + 8, pl.ds(v * 16, 16)]
                msgb[i, pl.ds(v * 16, 16)] = a + b
            return carry
        lax.fori_loop(0, 8, addr, 0)
        pltpu.sync_copy(msgb.at[pl.ds(0, 8)],
                        hh_ref.at[pl.ds(nrow + r0, 8)])

    def hloop(i, carry):
        hpiece(rbase + i * 8)
        return carry
    lax.fori_loop(0, STR // 8, hloop, 0)

    @pl.when(t == NS - 1)
    def _():
        hpiece(NS * STR)
        hpiece(NS * STR + 8)

    plsc.subcore_barrier()

    zero_agg()
    plsc.subcore_barrier()

    def chunk2(m, carry):
        j = m * NS + t

        @pl.when(j < ECH)
        def _():
            stage_edges(j)

            def grp16(g, c2):
                pltpu.async_copy(
                    hh_ref.at[srcb.at[pl.ds(g * 16, 16)]], msgb,
                    sem0).wait()
                pltpu.sync_copy(msgb, agg_sh.at[sidx.at[g]], add=True)
                return c2
            lax.fori_loop(0, 8, grp16, 0)
        return carry

    lax.fori_loop(0, EPW3, chunk2, 0)

    plsc.subcore_barrier()

    def opiece(r0):
        pltpu.sync_copy(agg_sh.at[pl.ds(r0, 16)], msgb)
        pltpu.sync_copy(msgb, a2_ref.at[pl.ds(nrow + r0, 16)])

    def oloop(i, carry):
        opiece(rbase + i * 16)
        return carry
    lax.fori_loop(0, STR // 16, oloop, 0)

    @pl.when(t == NS - 1)
    def _():
        opiece(NS * STR)


def _out_body(a2_ref, h_ref, rw_ref, rb_ref, r2_ref, out_ref):
    acc = jnp.dot(a2_ref[0], rw_ref[0], preferred_element_type=jnp.float32)
    acc = acc + jnp.dot(a2_ref[1], rw_ref[1], preferred_element_type=jnp.float32)
    acc = acc + jnp.dot(h_ref[0], r2_ref[0], preferred_element_type=jnp.float32)
    acc = acc + jnp.dot(h_ref[1], r2_ref[1], preferred_element_type=jnp.float32)
    out_ref[...] = acc + rb_ref[...]


def _compute_out(agg2, hh, rel_w2, rel_b, root2_w2):
    return pl.pallas_call(
        _out_body,
        grid=(N // BM,),
        in_specs=[pl.BlockSpec((NC, BM, HHALF), lambda i: (0, i, 0)),
                  pl.BlockSpec((NC, BM, HHALF), lambda i: (0, i, 0)),
                  pl.BlockSpec((NC, HHALF, H2), lambda i: (0, 0, 0)),
                  pl.BlockSpec((1, H2), lambda i: (0, 0)),
                  pl.BlockSpec((NC, HHALF, H2), lambda i: (0, 0, 0))],
        out_specs=pl.BlockSpec((BM, H2), lambda i: (i, 0)),
        out_shape=jax.ShapeDtypeStruct((N, H2), jnp.float32),
    )(agg2, hh, rel_w2, rel_b, root2_w2)


def kernel(node_features, edge_index, edge_norm, edge_type, comp, basis,
           root1_w, root1_b, rel_w, rel_b, root2_w):
    x = node_features
    eflat = edge_index.reshape(2 * E)

    xb, xr, compw = _compute_xb(x, basis, root1_w, root1_b.reshape(1, H1),
                                comp)
    inv = _hist_kernel(eflat, edge_norm)
    ce = _ce_kernel(eflat, edge_norm, inv, compw)
    hh, agg2 = _msg_kernel(xb.reshape(NC * N, XBW), eflat, ce, xr)
    return _compute_out(agg2.reshape(NC, N, HHALF), hh.reshape(NC, N, HHALF),
                        rel_w.reshape(NC, HHALF, H2), rel_b.reshape(1, H2),
                        root2_w.reshape(NC, HHALF, H2))
